# Initial kernel scaffold; baseline (speedup 1.0000x reference)
import functools

import jax
import jax.numpy as jnp
from jax import lax
from jax.experimental import pallas as pl
from jax.experimental.pallas import tpu as pltpu
from jax.experimental.pallas import tpu_sc as plsc

_N = 10000
_E = 320000
_D = 128

_NC = 2
_NS = 16
_NW = _NC * _NS
_EPW = _E // _NW
_CH = 80
_NCHUNK = _EPW // _CH
_RPS = _N // _NS


def _sc_agg_body(with_counts, x_hbm, src_hbm, dst_hbm, z_hbm, zc_hbm,
                 ones_hbm, sum_out, cnt_out, src_v, dst_v, rows_v, ones_v,
                 ssum, scnt, sem):
    c = lax.axis_index("c")
    s = lax.axis_index("s")
    wid = s * _NC + c

    pltpu.sync_copy(src_hbm.at[wid], src_v)
    pltpu.sync_copy(dst_hbm.at[wid], dst_v)
    if with_counts:
        pltpu.sync_copy(ones_hbm, ones_v)

    r0 = s * _RPS
    pltpu.sync_copy(z_hbm, ssum.at[pl.ds(r0, _RPS)])
    if with_counts:
        pltpu.sync_copy(zc_hbm, scnt.at[pl.ds(r0, _RPS)])
    plsc.subcore_barrier()

    def chunk(j, carry):
        pltpu.async_copy(x_hbm.at[src_v.at[j]], rows_v, sem).wait()
        pltpu.sync_copy(rows_v, ssum.at[dst_v.at[j]], add=True)
        if with_counts:
            pltpu.sync_copy(ones_v, scnt.at[dst_v.at[j]], add=True)
        return carry

    lax.fori_loop(0, _NCHUNK, chunk, 0)
    plsc.subcore_barrier()

    pltpu.sync_copy(ssum.at[pl.ds(r0, _RPS)], sum_out.at[c, pl.ds(r0, _RPS)])
    if with_counts:
        pltpu.sync_copy(scnt.at[pl.ds(r0, _RPS)],
                        cnt_out.at[c, pl.ds(r0, _RPS)])


def _sc_agg_body_nc(x_hbm, src_hbm, dst_hbm, z_hbm, zc_hbm, ones_hbm,
                    sum_out, src_v, dst_v, rows_v, ones_v, ssum, scnt, sem):
    _sc_agg_body(False, x_hbm, src_hbm, dst_hbm, z_hbm, zc_hbm, ones_hbm,
                 sum_out, None, src_v, dst_v, rows_v, ones_v, ssum, scnt, sem)


def _make_sc_agg(with_counts):
    mesh = plsc.VectorSubcoreMesh(core_axis_name="c", subcore_axis_name="s")
    out_type = [jax.ShapeDtypeStruct((_NC, _N, _D), jnp.float32)]
    if with_counts:
        out_type.append(jax.ShapeDtypeStruct((_NC, _N, 16), jnp.float32))
    scratch = [
        pltpu.VMEM((_NCHUNK, _CH), jnp.int32),
        pltpu.VMEM((_NCHUNK, _CH), jnp.int32),
        pltpu.VMEM((_CH, _D), jnp.float32),
        pltpu.VMEM((_CH, 16), jnp.float32),
        pltpu.VMEM_SHARED((_N, _D), jnp.float32),
        pltpu.VMEM_SHARED((_N, 16), jnp.float32),
        pltpu.SemaphoreType.DMA,
    ]

    body = (functools.partial(_sc_agg_body, True) if with_counts
            else _sc_agg_body_nc)
    kern = pl.kernel(body, mesh=mesh, out_type=out_type,
                     scratch_types=scratch)

    def run(x, src3, dst3):
        z = jnp.zeros((_RPS, _D), jnp.float32)
        zc = jnp.zeros((_RPS, 16), jnp.float32)
        ones = jnp.ones((_CH, 16), jnp.float32)
        return kern(x, src3, dst3, z, zc, ones)

    return run


def _layer_body(p_ref, c_ref, x_ref, wl_ref, bl_ref, wr_ref, o_ref):
    cnt = c_ref[0, :, 0:1] + c_ref[1, :, 0:1]
    mean = (p_ref[0] + p_ref[1]) / jnp.maximum(cnt, 1.0)
    h = jnp.dot(mean, wl_ref[...], preferred_element_type=jnp.float32)
    h = h + jnp.dot(x_ref[...], wr_ref[...], preferred_element_type=jnp.float32)
    h = h + bl_ref[...]
    o_ref[...] = jnp.maximum(h, 0.0)


def _final_body(p_ref, c_ref, x_ref, wl_ref, bl_ref, wr_ref, wo_ref, bo_ref,
                o_ref):
    cnt = c_ref[0, :, 0:1] + c_ref[1, :, 0:1]
    mean = (p_ref[0] + p_ref[1]) / jnp.maximum(cnt, 1.0)
    h = jnp.dot(mean, wl_ref[...], preferred_element_type=jnp.float32)
    h = h + jnp.dot(x_ref[...], wr_ref[...], preferred_element_type=jnp.float32)
    h = jnp.maximum(h + bl_ref[...], 0.0)
    o_ref[...] = (jnp.dot(h, wo_ref[...], preferred_element_type=jnp.float32)
                  + bo_ref[...])


_TC_R = 1000


def _tc_layer(P, C2, x, Wl, bl, Wr):
    return pl.pallas_call(
        _layer_body,
        grid=(_N // _TC_R,),
        in_specs=[
            pl.BlockSpec((_NC, _TC_R, _D), lambda i: (0, i, 0)),
            pl.BlockSpec((_NC, _TC_R, 16), lambda i: (0, i, 0)),
            pl.BlockSpec((_TC_R, _D), lambda i: (i, 0)),
            pl.BlockSpec((_D, _D), lambda i: (0, 0)),
            pl.BlockSpec((1, _D), lambda i: (0, 0)),
            pl.BlockSpec((_D, _D), lambda i: (0, 0)),
        ],
        out_specs=pl.BlockSpec((_TC_R, _D), lambda i: (i, 0)),
        out_shape=jax.ShapeDtypeStruct((_N, _D), jnp.float32),
    )(P, C2, x, Wl, bl.reshape(1, _D), Wr)


def _tc_final(P, C2, x, Wl, bl, Wr, Wo, bo):
    return pl.pallas_call(
        _final_body,
        grid=(_N // _TC_R,),
        in_specs=[
            pl.BlockSpec((_NC, _TC_R, _D), lambda i: (0, i, 0)),
            pl.BlockSpec((_NC, _TC_R, 16), lambda i: (0, i, 0)),
            pl.BlockSpec((_TC_R, _D), lambda i: (i, 0)),
            pl.BlockSpec((_D, _D), lambda i: (0, 0)),
            pl.BlockSpec((1, _D), lambda i: (0, 0)),
            pl.BlockSpec((_D, _D), lambda i: (0, 0)),
            pl.BlockSpec((_D, _D), lambda i: (0, 0)),
            pl.BlockSpec((1, _D), lambda i: (0, 0)),
        ],
        out_specs=pl.BlockSpec((_TC_R, _D), lambda i: (i, 0)),
        out_shape=jax.ShapeDtypeStruct((_N, _D), jnp.float32),
    )(P, C2, x, Wl, bl.reshape(1, _D), Wr, Wo, bo.reshape(1, _D))


_sc_agg_counts = _make_sc_agg(True)
_sc_agg_plain = _make_sc_agg(False)


def kernel(x, edge_index, Wl0, bl0, Wr0, Wl1, bl1, Wr1, Wlin, blin):
    src3 = edge_index[0].reshape(_NW, _NCHUNK, _CH)
    dst3 = edge_index[1].reshape(_NW, _NCHUNK, _CH)
    P0, C2 = _sc_agg_counts(x, src3, dst3)
    h1 = _tc_layer(P0, C2, x, Wl0, bl0, Wr0)
    P1 = _sc_agg_plain(h1, src3, dst3)
    return _tc_final(P1, C2, h1, Wl1, bl1, Wr1, Wlin, blin)


# speedup vs baseline: 5.6285x; 5.6285x over previous
import jax
import jax.numpy as jnp
from jax import lax
from jax.experimental import pallas as pl
from jax.experimental.pallas import tpu as pltpu
from jax.experimental.pallas import tpu_sc as plsc

_N = 10000
_E = 320000
_D = 128

_NC = 2
_NS = 16
_NW = _NC * _NS
_EPW = _E // _NW
_CH = 80
_NCHUNK = _EPW // _CH
_NP = 10240
_RPS = _NP // _NS


def _sc_pass_body(table_hbm, edges_hbm, z_hbm, sum_out,
                  idx_v, rows_v, ssum, sem):
    c = lax.axis_index("c")
    s = lax.axis_index("s")
    wid = s * _NC + c

    pltpu.sync_copy(edges_hbm.at[wid], idx_v)
    src_v = idx_v.at[0]
    dst_v = idx_v.at[1]

    r0 = s * _RPS
    pltpu.sync_copy(z_hbm, ssum.at[pl.ds(r0, _RPS)])
    plsc.subcore_barrier()

    def chunk(j, carry):
        pltpu.async_copy(table_hbm.at[src_v.at[j]], rows_v, sem).wait()
        pltpu.sync_copy(rows_v, ssum.at[dst_v.at[j]], add=True)
        return carry

    lax.fori_loop(0, _NCHUNK, chunk, 0)
    plsc.subcore_barrier()

    pltpu.sync_copy(ssum.at[pl.ds(r0, _RPS)], sum_out.at[c, pl.ds(r0, _RPS)])


_SC_MESH = plsc.VectorSubcoreMesh(core_axis_name="c", subcore_axis_name="s")

_sc_pass_kern = pl.kernel(
    _sc_pass_body,
    mesh=_SC_MESH,
    out_type=jax.ShapeDtypeStruct((_NC, _NP, _D), jnp.float32),
    scratch_types=[
        pltpu.VMEM((2, _NCHUNK, _CH), jnp.int32),
        pltpu.VMEM((_CH, _D), jnp.float32),
        pltpu.VMEM_SHARED((_NP, _D), jnp.float32),
        pltpu.SemaphoreType.DMA,
    ],
)


def _sc_pass(table, edges4, z):
    return _sc_pass_kern(table, edges4, z)


def _layer_body(p_ref, c_ref, x_ref, wl_ref, bl_ref, wr_ref, o_ref):
    cnt = c_ref[0] + c_ref[1]
    mean = (p_ref[0] + p_ref[1]) / jnp.maximum(cnt, 1.0)
    h = jnp.dot(mean, wl_ref[...], preferred_element_type=jnp.float32)
    h = h + jnp.dot(x_ref[...], wr_ref[...], preferred_element_type=jnp.float32)
    h = h + bl_ref[...]
    o_ref[...] = jnp.maximum(h, 0.0)


def _final_body(p_ref, c_ref, x_ref, wl_ref, bl_ref, wr_ref, wo_ref, bo_ref,
                o_ref):
    cnt = c_ref[0] + c_ref[1]
    mean = (p_ref[0] + p_ref[1]) / jnp.maximum(cnt, 1.0)
    h = jnp.dot(mean, wl_ref[...], preferred_element_type=jnp.float32)
    h = h + jnp.dot(x_ref[...], wr_ref[...], preferred_element_type=jnp.float32)
    h = jnp.maximum(h + bl_ref[...], 0.0)
    o_ref[...] = (jnp.dot(h, wo_ref[...], preferred_element_type=jnp.float32)
                  + bo_ref[...])


_TC_R = 1000


def _tc_layer(P, C2, x, Wl, bl, Wr):
    return pl.pallas_call(
        _layer_body,
        grid=(_N // _TC_R,),
        in_specs=[
            pl.BlockSpec((_NC, _TC_R, _D), lambda i: (0, i, 0)),
            pl.BlockSpec((_NC, _TC_R, _D), lambda i: (0, i, 0)),
            pl.BlockSpec((_TC_R, _D), lambda i: (i, 0)),
            pl.BlockSpec((_D, _D), lambda i: (0, 0)),
            pl.BlockSpec((1, _D), lambda i: (0, 0)),
            pl.BlockSpec((_D, _D), lambda i: (0, 0)),
        ],
        out_specs=pl.BlockSpec((_TC_R, _D), lambda i: (i, 0)),
        out_shape=jax.ShapeDtypeStruct((_N, _D), jnp.float32),
    )(P, C2, x, Wl, bl.reshape(1, _D), Wr)


def _tc_final(P, C2, x, Wl, bl, Wr, Wo, bo):
    return pl.pallas_call(
        _final_body,
        grid=(_N // _TC_R,),
        in_specs=[
            pl.BlockSpec((_NC, _TC_R, _D), lambda i: (0, i, 0)),
            pl.BlockSpec((_NC, _TC_R, _D), lambda i: (0, i, 0)),
            pl.BlockSpec((_TC_R, _D), lambda i: (i, 0)),
            pl.BlockSpec((_D, _D), lambda i: (0, 0)),
            pl.BlockSpec((1, _D), lambda i: (0, 0)),
            pl.BlockSpec((_D, _D), lambda i: (0, 0)),
            pl.BlockSpec((_D, _D), lambda i: (0, 0)),
            pl.BlockSpec((1, _D), lambda i: (0, 0)),
        ],
        out_specs=pl.BlockSpec((_TC_R, _D), lambda i: (i, 0)),
        out_shape=jax.ShapeDtypeStruct((_N, _D), jnp.float32),
    )(P, C2, x, Wl, bl.reshape(1, _D), Wr, Wo, bo.reshape(1, _D))


def kernel(x, edge_index, Wl0, bl0, Wr0, Wl1, bl1, Wr1, Wlin, blin):
    edges4 = edge_index.reshape(2, _NW, _NCHUNK, _CH).transpose(1, 0, 2, 3)
    seq = jnp.arange(_EPW, dtype=jnp.int32).reshape(1, _NCHUNK, _CH)
    edges4c = jnp.concatenate(
        [jnp.broadcast_to(seq, (_NW, 1, _NCHUNK, _CH)), edges4[:, 1:2]], axis=1)
    ones_tab = jnp.ones((_N, _D), jnp.float32)
    z = jnp.zeros((_RPS, _D), jnp.float32)
    C2 = _sc_pass(ones_tab, edges4c, z)
    z0 = C2[0, :_RPS] * 0.0
    P0 = _sc_pass(x, edges4, z0)
    h1 = _tc_layer(P0, C2, x, Wl0, bl0, Wr0)
    P1 = _sc_pass(h1, edges4, z0)
    return _tc_final(P1, C2, h1, Wl1, bl1, Wr1, Wlin, blin)
